# baseline (speedup 1.0000x reference)
import functools

import jax
import jax.numpy as jnp
from jax import lax
from jax.experimental import pallas as pl
from jax.experimental.pallas import tpu as pltpu
from jax.experimental.pallas import tpu_sc as plsc

B, S = 16, 2048
E = B * S
NC, NS = 2, 16
NW = NC * NS
PER_W = E // NW
NCH = 8
CH = PER_W // NCH
NBUF = 7
DEPTH = 5
NREP = 8
PHONE_D, MIDI_D = 128, 64
F0_D, UNV_D = 64, 16

SBLK = 512


def _sc_body(ph_idx_hbm, pt_hbm, out_ph_hbm, idx_v, buf, gsem, wsem):
    wid = lax.axis_index("s") * NC + lax.axis_index("c")
    base = wid * PER_W

    pltpu.sync_copy(ph_idx_hbm.at[wid], idx_v)

    gathers = [
        pltpu.make_async_copy(pt_hbm.at[wid % NREP].at[idx_v.at[c]],
                              buf.at[c % NBUF], gsem)
        for c in range(NCH)
    ]
    writes = [
        pltpu.make_async_copy(
            buf.at[c % NBUF],
            out_ph_hbm.at[pl.ds(base + c * CH, CH)],
            wsem.at[c % NBUF],
        )
        for c in range(NCH)
    ]

    for c in range(DEPTH):
        gathers[c].start()
    for c in range(NCH):
        gathers[c].wait()
        writes[c].start()
        if c + DEPTH < NCH:
            if c + DEPTH - NBUF >= 0:
                writes[c + DEPTH - NBUF].wait()
            gathers[c + DEPTH].start()
    for c in range(NCH - NBUF, NCH):
        writes[c].wait()


@jax.jit
def _sc_phone(ph_idx, phone_table_rep):
    mesh = plsc.VectorSubcoreMesh(core_axis_name="c", subcore_axis_name="s")
    return pl.kernel(
        _sc_body,
        out_type=jax.ShapeDtypeStruct((E, PHONE_D), jnp.float32),
        mesh=mesh,
        scratch_types=[
            pltpu.VMEM((NCH, CH), jnp.int32),
            pltpu.VMEM((NBUF, CH, PHONE_D), jnp.float32),
            pltpu.SemaphoreType.DMA,
            pltpu.SemaphoreType.DMA((NBUF,)),
        ],
        compiler_params=pltpu.CompilerParams(use_tc_tiling_on_sc=True),
    )(ph_idx, phone_table_rep)


def _tc_body(md_ref, f0_ref, un_ref, mtt_ref, wf_ref, bf_ref, wu_ref, bu_ref,
             om_ref, of_ref, ou_ref):
    f0v = f0_ref[...]
    wf = wf_ref[...]
    bf = bf_ref[...]
    of_ref[...] = f0v[:, None, :] * wf[None, :, :] + bf[None, :, :]
    unv = un_ref[...]
    wu = wu_ref[...]
    bu = bu_ref[...]
    ou_ref[...] = unv[:, None, :] * wu[None, :, :] + bu[None, :, :]

    mtt = mtt_ref[...]
    md = md_ref[...]
    iota = lax.broadcasted_iota(jnp.int32, (128, SBLK), 0)
    for b in range(B):
        oh = (md[b][None, :] == iota).astype(jnp.float32)
        om_ref[b] = jnp.dot(mtt, oh, preferred_element_type=jnp.float32)


@jax.jit
def _tc_rest(md2, f02, un2, mtt, wf, bf, wu, bu):
    grid = S // SBLK
    blk_in = pl.BlockSpec((B, SBLK), lambda i: (0, i))
    full = lambda shape: pl.BlockSpec(shape, lambda i: (0,) * len(shape))
    out_spec = lambda d: pl.BlockSpec((B, d, SBLK), lambda i: (0, 0, i))
    return pl.pallas_call(
        _tc_body,
        grid=(grid,),
        in_specs=[blk_in, blk_in, blk_in, full((MIDI_D, 128)),
                  full((F0_D, 1)), full((F0_D, 1)),
                  full((UNV_D, 1)), full((UNV_D, 1))],
        out_specs=[out_spec(MIDI_D), out_spec(F0_D), out_spec(UNV_D)],
        out_shape=[jax.ShapeDtypeStruct((B, MIDI_D, S), jnp.float32),
                   jax.ShapeDtypeStruct((B, F0_D, S), jnp.float32),
                   jax.ShapeDtypeStruct((B, UNV_D, S), jnp.float32)],
    )(md2, f02, un2, mtt, wf, bf, wu, bu)


def kernel(f0, phone_label, phone_duration, midi_label, unvoiced_flag,
           W_f0, b_f0, phone_table, midi_table, W_unv, b_unv):
    ph_idx = phone_label.astype(jnp.int32).reshape(NW, NCH, CH)
    pt_rep = jnp.broadcast_to(phone_table[None], (NREP,) + phone_table.shape)
    md2 = midi_label.astype(jnp.int32)
    f02 = f0.reshape(B, S)
    un2 = unvoiced_flag
    mtt = midi_table.T
    wf = W_f0.reshape(F0_D, 1)
    bf = b_f0.reshape(F0_D, 1)
    wu = W_unv.reshape(UNV_D, 1)
    bu = b_unv.reshape(UNV_D, 1)

    op = _sc_phone(ph_idx, pt_rep)
    om_t, of_t, ou_t = _tc_rest(md2, f02, un2, mtt, wf, bf, wu, bu)
    return (of_t.transpose(0, 2, 1), op.reshape(B, S, PHONE_D),
            om_t.transpose(0, 2, 1), ou_t.transpose(0, 2, 1))


# speedup vs baseline: 1.1050x; 1.1050x over previous
import functools

import jax
import jax.numpy as jnp
from jax import lax
from jax.experimental import pallas as pl
from jax.experimental.pallas import tpu as pltpu
from jax.experimental.pallas import tpu_sc as plsc

B, S = 16, 2048
E = B * S
NC, NS = 2, 16
NW = NC * NS
PER_W = E // NW
NCH = 8
CH = PER_W // NCH
NBUF = 7
DEPTH = 5
NREP = NW
PHONE_D, MIDI_D = 128, 64
F0_D, UNV_D = 64, 16

SBLK = 512


def _sc_body(ph_idx_hbm, pt_hbm, out_ph_hbm, idx_v, buf, gsem, wsem):
    wid = lax.axis_index("s") * NC + lax.axis_index("c")
    base = wid * PER_W

    pltpu.sync_copy(ph_idx_hbm.at[wid], idx_v)

    gathers = [
        pltpu.make_async_copy(pt_hbm.at[wid % NREP].at[idx_v.at[c]],
                              buf.at[c % NBUF], gsem)
        for c in range(NCH)
    ]
    writes = [
        pltpu.make_async_copy(
            buf.at[c % NBUF],
            out_ph_hbm.at[pl.ds(base + c * CH, CH)],
            wsem.at[c % NBUF],
        )
        for c in range(NCH)
    ]

    for c in range(DEPTH):
        gathers[c].start()
    for c in range(NCH):
        gathers[c].wait()
        writes[c].start()
        if c + DEPTH < NCH:
            if c + DEPTH - NBUF >= 0:
                writes[c + DEPTH - NBUF].wait()
            gathers[c + DEPTH].start()
    for c in range(NCH - NBUF, NCH):
        writes[c].wait()


@jax.jit
def _sc_phone(ph_idx, phone_table_rep):
    mesh = plsc.VectorSubcoreMesh(core_axis_name="c", subcore_axis_name="s")
    return pl.kernel(
        _sc_body,
        out_type=jax.ShapeDtypeStruct((E, PHONE_D), jnp.float32),
        mesh=mesh,
        scratch_types=[
            pltpu.VMEM((NCH, CH), jnp.int32),
            pltpu.VMEM((NBUF, CH, PHONE_D), jnp.float32),
            pltpu.SemaphoreType.DMA,
            pltpu.SemaphoreType.DMA((NBUF,)),
        ],
        compiler_params=pltpu.CompilerParams(use_tc_tiling_on_sc=True),
    )(ph_idx, phone_table_rep)


def _tc_body(md_ref, f0_ref, un_ref, mtt_ref, wf_ref, bf_ref, wu_ref, bu_ref,
             om_ref, of_ref, ou_ref):
    f0v = f0_ref[...]
    wf = wf_ref[...]
    bf = bf_ref[...]
    of_ref[...] = f0v[:, None, :] * wf[None, :, :] + bf[None, :, :]
    unv = un_ref[...]
    wu = wu_ref[...]
    bu = bu_ref[...]
    ou_ref[...] = unv[:, None, :] * wu[None, :, :] + bu[None, :, :]

    mtt = mtt_ref[...]
    md = md_ref[...]
    iota = lax.broadcasted_iota(jnp.int32, (128, SBLK), 0)
    for b in range(B):
        oh = (md[b][None, :] == iota).astype(jnp.float32)
        om_ref[b] = jnp.dot(mtt, oh, preferred_element_type=jnp.float32)


@jax.jit
def _tc_rest(md2, f02, un2, mtt, wf, bf, wu, bu):
    grid = S // SBLK
    blk_in = pl.BlockSpec((B, SBLK), lambda i: (0, i))
    full = lambda shape: pl.BlockSpec(shape, lambda i: (0,) * len(shape))
    out_spec = lambda d: pl.BlockSpec((B, d, SBLK), lambda i: (0, 0, i))
    return pl.pallas_call(
        _tc_body,
        grid=(grid,),
        in_specs=[blk_in, blk_in, blk_in, full((MIDI_D, 128)),
                  full((F0_D, 1)), full((F0_D, 1)),
                  full((UNV_D, 1)), full((UNV_D, 1))],
        out_specs=[out_spec(MIDI_D), out_spec(F0_D), out_spec(UNV_D)],
        out_shape=[jax.ShapeDtypeStruct((B, MIDI_D, S), jnp.float32),
                   jax.ShapeDtypeStruct((B, F0_D, S), jnp.float32),
                   jax.ShapeDtypeStruct((B, UNV_D, S), jnp.float32)],
    )(md2, f02, un2, mtt, wf, bf, wu, bu)


def kernel(f0, phone_label, phone_duration, midi_label, unvoiced_flag,
           W_f0, b_f0, phone_table, midi_table, W_unv, b_unv):
    ph_idx = phone_label.astype(jnp.int32).reshape(NW, NCH, CH)
    pt_rep = jnp.broadcast_to(phone_table[None], (NREP,) + phone_table.shape)
    md2 = midi_label.astype(jnp.int32)
    f02 = f0.reshape(B, S)
    un2 = unvoiced_flag
    mtt = midi_table.T
    wf = W_f0.reshape(F0_D, 1)
    bf = b_f0.reshape(F0_D, 1)
    wu = W_unv.reshape(UNV_D, 1)
    bu = b_unv.reshape(UNV_D, 1)

    op = _sc_phone(ph_idx, pt_rep)
    om_t, of_t, ou_t = _tc_rest(md2, f02, un2, mtt, wf, bf, wu, bu)
    return (of_t.transpose(0, 2, 1), op.reshape(B, S, PHONE_D),
            om_t.transpose(0, 2, 1), ou_t.transpose(0, 2, 1))


# speedup vs baseline: 1.1299x; 1.0226x over previous
import functools

import jax
import jax.numpy as jnp
from jax import lax
from jax.experimental import pallas as pl
from jax.experimental.pallas import tpu as pltpu
from jax.experimental.pallas import tpu_sc as plsc

B, S = 16, 2048
E = B * S
NC, NS = 2, 16
NW = NC * NS
PER_W = E // NW
NCH = 8
CH = PER_W // NCH
NBUF = 7
DEPTH = 5
NREP = NW
PHONE_D, MIDI_D = 128, 64
F0_D, UNV_D = 64, 16

SBLK = 512


def _sc_body(ph_lbl_hbm, pt_hbm, out_ph_hbm, idx_v, buf, gsem, wsem):
    wid = lax.axis_index("s") * NC + lax.axis_index("c")
    base = wid * PER_W
    b = wid // (S // PER_W)
    s0 = (wid % (S // PER_W)) * PER_W

    pltpu.sync_copy(ph_lbl_hbm.at[b, pl.ds(s0, PER_W)], idx_v)

    gathers = [
        pltpu.make_async_copy(
            pt_hbm.at[wid % NREP].at[idx_v.at[pl.ds(c * CH, CH)]],
            buf.at[c % NBUF], gsem)
        for c in range(NCH)
    ]
    writes = [
        pltpu.make_async_copy(
            buf.at[c % NBUF],
            out_ph_hbm.at[pl.ds(base + c * CH, CH)],
            wsem.at[c % NBUF],
        )
        for c in range(NCH)
    ]

    for c in range(DEPTH):
        gathers[c].start()
    for c in range(NCH):
        gathers[c].wait()
        writes[c].start()
        if c + DEPTH < NCH:
            if c + DEPTH - NBUF >= 0:
                writes[c + DEPTH - NBUF].wait()
            gathers[c + DEPTH].start()
    for c in range(NCH - NBUF, NCH):
        writes[c].wait()


@jax.jit
def _sc_phone(ph_lbl, phone_table_rep):
    mesh = plsc.VectorSubcoreMesh(core_axis_name="c", subcore_axis_name="s")
    return pl.kernel(
        _sc_body,
        out_type=jax.ShapeDtypeStruct((E, PHONE_D), jnp.float32),
        mesh=mesh,
        scratch_types=[
            pltpu.VMEM((PER_W,), jnp.int32),
            pltpu.VMEM((NBUF, CH, PHONE_D), jnp.float32),
            pltpu.SemaphoreType.DMA,
            pltpu.SemaphoreType.DMA((NBUF,)),
        ],
        compiler_params=pltpu.CompilerParams(use_tc_tiling_on_sc=True),
    )(ph_lbl, phone_table_rep)


def _tc_body(md_ref, f0_ref, un_ref, mtt_ref, wb_ref, om_ref, of_ref, ou_ref):
    wb = wb_ref[...]
    f0v = f0_ref[...]
    wf = wb[:F0_D, 0:1]
    bf = wb[:F0_D, 1:2]
    of_ref[...] = f0v[:, None, :] * wf[None, :, :] + bf[None, :, :]
    unv = un_ref[...]
    wu = wb[F0_D:, 0:1]
    bu = wb[F0_D:, 1:2]
    ou_ref[...] = unv[:, None, :] * wu[None, :, :] + bu[None, :, :]

    mtt = mtt_ref[...]
    md = md_ref[...]
    iota = lax.broadcasted_iota(jnp.int32, (128, SBLK), 0)
    for b in range(B):
        oh = (md[b][None, :] == iota).astype(jnp.float32)
        om_ref[b] = jnp.dot(mtt, oh, preferred_element_type=jnp.float32)


@jax.jit
def _tc_rest(md2, f02, un2, mtt, wb):
    grid = S // SBLK
    blk_in = pl.BlockSpec((B, SBLK), lambda i: (0, i))
    full = lambda shape: pl.BlockSpec(shape, lambda i: (0,) * len(shape))
    out_spec = lambda d: pl.BlockSpec((B, d, SBLK), lambda i: (0, 0, i))
    return pl.pallas_call(
        _tc_body,
        grid=(grid,),
        in_specs=[blk_in, blk_in, blk_in, full((MIDI_D, 128)),
                  full((F0_D + UNV_D, 2))],
        out_specs=[out_spec(MIDI_D), out_spec(F0_D), out_spec(UNV_D)],
        out_shape=[jax.ShapeDtypeStruct((B, MIDI_D, S), jnp.float32),
                   jax.ShapeDtypeStruct((B, F0_D, S), jnp.float32),
                   jax.ShapeDtypeStruct((B, UNV_D, S), jnp.float32)],
    )(md2, f02, un2, mtt, wb)


def kernel(f0, phone_label, phone_duration, midi_label, unvoiced_flag,
           W_f0, b_f0, phone_table, midi_table, W_unv, b_unv):
    ph_lbl = phone_label.astype(jnp.int32)
    pt_rep = jnp.broadcast_to(phone_table[None], (NREP,) + phone_table.shape)
    md2 = midi_label.astype(jnp.int32)
    f02 = f0.reshape(B, S)
    un2 = unvoiced_flag
    mtt = midi_table.T
    wb = jnp.concatenate(
        [jnp.stack([W_f0[:, 0], b_f0], axis=1),
         jnp.stack([W_unv[:, 0], b_unv], axis=1)], axis=0)

    op = _sc_phone(ph_lbl, pt_rep)
    om_t, of_t, ou_t = _tc_rest(md2, f02, un2, mtt, wb)
    return (of_t.transpose(0, 2, 1), op.reshape(B, S, PHONE_D),
            om_t.transpose(0, 2, 1), ou_t.transpose(0, 2, 1))
